# baseline (speedup 1.0000x reference)
import functools

import jax
import jax.numpy as jnp
from jax import lax
from jax.experimental import pallas as pl
from jax.experimental.pallas import tpu as pltpu
from jax.experimental.pallas import tpu_sc as plsc

N = 10000
E = 320000
H = 128
G = 64


def _lin_body(x_ref, w_ref, b_ref, o_ref):
    o_ref[...] = (
        jnp.dot(x_ref[...], w_ref[...], preferred_element_type=jnp.float32, precision=lax.Precision.HIGHEST)
        + b_ref[...]
    ).astype(o_ref.dtype)


def _linear(x, w, b2d, block_rows, out_dtype=jnp.float32):
    n, k = x.shape
    m = w.shape[1]
    return pl.pallas_call(
        _lin_body,
        grid=(n // block_rows,),
        in_specs=[
            pl.BlockSpec((block_rows, k), lambda i: (i, 0)),
            pl.BlockSpec((k, m), lambda i: (0, 0)),
            pl.BlockSpec((1, m), lambda i: (0, 0)),
        ],
        out_specs=pl.BlockSpec((block_rows, m), lambda i: (i, 0)),
        out_shape=jax.ShapeDtypeStruct((n, m), out_dtype),
    )(x, w, b2d)


_DN0 = (((0,), (0,)), ((), ()))


def _segsum(ot_blk, v):
    return lax.dot_general(ot_blk, v, _DN0, preferred_element_type=jnp.float32, precision=lax.Precision.HIGHEST)


def _onehot_body(bc_ref, ot_ref, inv_ref):
    bc = bc_ref[...]
    ot = (
        jnp.broadcast_to(bc, (N, G)) == lax.broadcasted_iota(jnp.int32, (N, G), 1)
    ).astype(jnp.float32)
    ot_ref[...] = ot
    cnt = _segsum(ot, jnp.ones((N, H), jnp.float32))
    inv_ref[...] = 1.0 / jnp.maximum(cnt, 1.0)


def _build_onehot(batch):
    return pl.pallas_call(
        _onehot_body,
        in_specs=[pl.BlockSpec((N, 1), lambda: (0, 0))],
        out_specs=[
            pl.BlockSpec((N, G), lambda: (0, 0)),
            pl.BlockSpec((G, H), lambda: (0, 0)),
        ],
        out_shape=[
            jax.ShapeDtypeStruct((N, G), jnp.float32),
            jax.ShapeDtypeStruct((G, H), jnp.float32),
        ],
    )(batch.reshape(N, 1))


C = 80
NW = 32
EPW = E // NW
RPT = EPW // C
NG = 25
GRP = RPT // NG
RZ = 16
NZCH = N // RZ

@functools.lru_cache(maxsize=None)
def _get_mp_kernel():
    mesh = plsc.VectorSubcoreMesh(core_axis_name="c", subcore_axis_name="s")
    return functools.partial(
        pl.kernel,
        out_type=(
            jax.ShapeDtypeStruct((N, H), jnp.float32),
            jax.ShapeDtypeStruct((N, H), jnp.float32),
        ),
        mesh=mesh,
        scratch_types=[
            pltpu.VMEM((GRP, C), jnp.int32),
            pltpu.VMEM((GRP, C), jnp.int32),
            pltpu.VMEM((C, H), jnp.float32),
            pltpu.VMEM((C, H), jnp.float32),
            pltpu.VMEM((C, H // 2), jnp.int32),
            pltpu.VMEM((C, H // 2), jnp.int32),
            pltpu.VMEM((RZ, H), jnp.float32),
            pltpu.VMEM_SHARED((N, H), jnp.float32),
            pltpu.SemaphoreType.DMA,
            pltpu.SemaphoreType.DMA,
            pltpu.SemaphoreType.DMA,
            pltpu.SemaphoreType.DMA,
            pltpu.SemaphoreType.DMA,
            pltpu.SemaphoreType.DMA,
        ],
    )(_mp_body)


def _mp_body(h_hbm, ea_hbm, src_hbm, dst_hbm, out0, out1,
             srcv, dstv, hb0, hb1, eb0, eb1, zbuf, acc,
             gs0, gs1, es0, es1, ss0, ss1):
    c = lax.axis_index("c")
    s = lax.axis_index("s")
    wid = c * 16 + s

    def zrow(r, carry):
        for k in range(H // 16):
            zbuf[r, pl.ds(k * 16, 16)] = jnp.zeros((16,), jnp.float32)
        return carry

    lax.fori_loop(0, RZ, zrow, 0)
    for k in range((NZCH + 15) // 16):
        idx = k * 16 + s

        @pl.when(idx < NZCH)
        def _():
            pltpu.sync_copy(zbuf, acc.at[pl.ds(idx * RZ, RZ)])

    plsc.subcore_barrier()

    ebase = wid * EPW

    def group(g, carry):
        pltpu.sync_copy(src_hbm.at[wid, g], srcv)
        pltpu.sync_copy(dst_hbm.at[wid, g], dstv)
        eb_g = ebase + g * GRP * C
        pltpu.async_copy(h_hbm.at[srcv.at[0]], hb0, gs0)
        pltpu.async_copy(ea_hbm.at[pl.ds(eb_g, C)], eb0, es0)

        def step(t, cy):
            even = lax.rem(t, 2) == 0

            def stage(hbp, ebp, gsp, esp, ssp, hbn, ebn, gsn, esn, ssn):
                @pl.when(t + 1 < GRP)
                def _():
                    @pl.when(t >= 1)
                    def _():
                        pltpu.make_async_copy(
                            hbn, acc.at[dstv.at[t - 1]], ssn
                        ).wait()

                    pltpu.async_copy(h_hbm.at[srcv.at[t + 1]], hbn, gsn)
                    pltpu.async_copy(
                        ea_hbm.at[pl.ds(eb_g + (t + 1) * C, C)], ebn, esn
                    )

                pltpu.make_async_copy(h_hbm.at[srcv.at[t]], hbp, gsp).wait()
                pltpu.make_async_copy(
                    ea_hbm.at[pl.ds(eb_g + t * C, C)], ebp, esp
                ).wait()

                sixteen = jnp.full((16,), 16, jnp.int32)
                himask = jnp.full((16,), -65536, jnp.int32)

                def rowfn(r, cz):
                    for q in range(H // 32):
                        w = ebp[r, pl.ds(q * 16, 16)]
                        a = lax.bitcast_convert_type(lax.shift_left(w, sixteen), jnp.float32)
                        b = lax.bitcast_convert_type(lax.bitwise_and(w, himask), jnp.float32)
                        slo = pl.ds(q * 32, 16)
                        shi = pl.ds(q * 32 + 16, 16)
                        hbp[r, slo] = jnp.maximum(hbp[r, slo] + a, 0.0)
                        hbp[r, shi] = jnp.maximum(hbp[r, shi] + b, 0.0)
                    return cz

                lax.fori_loop(0, C, rowfn, 0)
                pltpu.async_copy(hbp, acc.at[dstv.at[t]], ssp, add=True)

            @pl.when(even)
            def _():
                stage(hb0, eb0, gs0, es0, ss0, hb1, eb1, gs1, es1, ss1)

            @pl.when(jnp.logical_not(even))
            def _():
                stage(hb1, eb1, gs1, es1, ss1, hb0, eb0, gs0, es0, ss0)

            return cy

        lax.fori_loop(0, GRP, step, 0)
        pltpu.make_async_copy(hb1, acc.at[dstv.at[GRP - 2]], ss1).wait()
        pltpu.make_async_copy(hb0, acc.at[dstv.at[GRP - 1]], ss0).wait()
        return carry

    lax.fori_loop(0, NG, group, 0)
    plsc.subcore_barrier()

    for k in range((NZCH + 15) // 16):
        idx = k * 16 + s

        @pl.when(idx < NZCH)
        def _():
            pltpu.sync_copy(acc.at[pl.ds(idx * RZ, RZ)], zbuf)

            @pl.when(c == 0)
            def _():
                pltpu.sync_copy(zbuf, out0.at[pl.ds(idx * RZ, RZ)])

            @pl.when(c == 1)
            def _():
                pltpu.sync_copy(zbuf, out1.at[pl.ds(idx * RZ, RZ)])


def _passA_body(h_ref, a0_ref, a1_ref, w1_ref, b1_ref, w2_ref, b2_ref, ot_ref,
                t_ref, p_ref, q_ref):
    i = pl.program_id(0)
    z0 = h_ref[...] + a0_ref[...] + a1_ref[...]
    u = jnp.maximum(
        jnp.dot(z0, w1_ref[...], preferred_element_type=jnp.float32, precision=lax.Precision.HIGHEST) + b1_ref[...],
        0.0,
    )
    t = jnp.dot(u, w2_ref[...], preferred_element_type=jnp.float32, precision=lax.Precision.HIGHEST) + b2_ref[...]
    t_ref[...] = t

    @pl.when(i == 0)
    def _():
        p_ref[...] = jnp.zeros_like(p_ref)
        q_ref[...] = jnp.zeros_like(q_ref)

    ob = ot_ref[...]
    p_ref[...] += _segsum(ob, t)
    q_ref[...] += _segsum(ob, t * t)


def _layer_passA(h, a0, a1, w1, b1, w2, b2, onehot_t):
    br = 1000
    return pl.pallas_call(
        _passA_body,
        grid=(N // br,),
        in_specs=[
            pl.BlockSpec((br, H), lambda i: (i, 0)),
            pl.BlockSpec((br, H), lambda i: (i, 0)),
            pl.BlockSpec((br, H), lambda i: (i, 0)),
            pl.BlockSpec((H, H), lambda i: (0, 0)),
            pl.BlockSpec((1, H), lambda i: (0, 0)),
            pl.BlockSpec((H, H), lambda i: (0, 0)),
            pl.BlockSpec((1, H), lambda i: (0, 0)),
            pl.BlockSpec((br, G), lambda i: (i, 0)),
        ],
        out_specs=[
            pl.BlockSpec((br, H), lambda i: (i, 0)),
            pl.BlockSpec((G, H), lambda i: (0, 0)),
            pl.BlockSpec((G, H), lambda i: (0, 0)),
        ],
        out_shape=[
            jax.ShapeDtypeStruct((N, H), jnp.float32),
            jax.ShapeDtypeStruct((G, H), jnp.float32),
            jax.ShapeDtypeStruct((G, H), jnp.float32),
        ],
    )(h, a0, a1, w1, b1, w2, b2, onehot_t)


def _passB_body(t_ref, hin_ref, ot_ref, p_ref, q_ref, inv_ref, gw_ref, gb_ref,
                ga_ref, out_ref):
    ga = ga_ref[...]
    s1 = p_ref[...] * inv_ref[...]
    s2 = q_ref[...] * inv_ref[...]
    var = s2 - (2.0 * ga - ga * ga) * s1 * s1
    otb = ot_ref[...]
    mean_b = jnp.dot(otb, s1, preferred_element_type=jnp.float32, precision=lax.Precision.HIGHEST)
    var_b = jnp.dot(otb, var, preferred_element_type=jnp.float32, precision=lax.Precision.HIGHEST)
    o = t_ref[...] - mean_b * ga
    z = gw_ref[...] * o * lax.rsqrt(var_b + 1e-5) + gb_ref[...]
    out_ref[...] = jnp.maximum(z, 0.0) + hin_ref[...]


def _layer_passB(t, h_in, ot, p, q, inv, gw, gb, ga):
    br = 1000
    return pl.pallas_call(
        _passB_body,
        grid=(N // br,),
        in_specs=[
            pl.BlockSpec((br, H), lambda i: (i, 0)),
            pl.BlockSpec((br, H), lambda i: (i, 0)),
            pl.BlockSpec((br, G), lambda i: (i, 0)),
            pl.BlockSpec((G, H), lambda i: (0, 0)),
            pl.BlockSpec((G, H), lambda i: (0, 0)),
            pl.BlockSpec((G, H), lambda i: (0, 0)),
            pl.BlockSpec((1, H), lambda i: (0, 0)),
            pl.BlockSpec((1, H), lambda i: (0, 0)),
            pl.BlockSpec((1, H), lambda i: (0, 0)),
        ],
        out_specs=pl.BlockSpec((br, H), lambda i: (i, 0)),
        out_shape=jax.ShapeDtypeStruct((N, H), jnp.float32),
    )(t, h_in, ot, p, q, inv, gw, gb, ga)


def _pool_body(h0_ref, h1_ref, h2_ref, h3_ref, ot_ref, g0_ref, g1_ref, g2_ref,
               g3_ref):
    i = pl.program_id(0)

    @pl.when(i == 0)
    def _():
        g0_ref[...] = jnp.zeros_like(g0_ref)
        g1_ref[...] = jnp.zeros_like(g1_ref)
        g2_ref[...] = jnp.zeros_like(g2_ref)
        g3_ref[...] = jnp.zeros_like(g3_ref)

    ob = ot_ref[...]
    g0_ref[...] += _segsum(ob, h0_ref[...])
    g1_ref[...] += _segsum(ob, h1_ref[...])
    g2_ref[...] += _segsum(ob, h2_ref[...])
    g3_ref[...] += _segsum(ob, h3_ref[...])


def _pool(h0, h1, h2, h3, onehot_t):
    br = 1000
    return pl.pallas_call(
        _pool_body,
        grid=(N // br,),
        in_specs=[pl.BlockSpec((br, H), lambda i: (i, 0))] * 4
        + [pl.BlockSpec((br, G), lambda i: (i, 0))],
        out_specs=[pl.BlockSpec((G, H), lambda i: (0, 0))] * 4,
        out_shape=[jax.ShapeDtypeStruct((G, H), jnp.float32)] * 4,
    )(h0, h1, h2, h3, onehot_t)


def _head_body(g0_ref, g1_ref, g2_ref, g3_ref, inv_ref, gf_ref, m00_ref,
               m01_ref, m02_ref, m03_ref, m0g_ref, mb0_ref, bn0g_ref, bn0b_ref,
               mw1_ref, mb1_ref, bn1g_ref, bn1b_ref, mw2_ref, mb2_ref, mw3_ref,
               mb3_ref, y_ref):
    s = (1.0 + 1e-5) ** -0.5
    inv = inv_ref[...]
    y = (
        jnp.dot(g0_ref[...] * inv, m00_ref[...], preferred_element_type=jnp.float32, precision=lax.Precision.HIGHEST)
        + jnp.dot(g1_ref[...] * inv, m01_ref[...], preferred_element_type=jnp.float32, precision=lax.Precision.HIGHEST)
        + jnp.dot(g2_ref[...] * inv, m02_ref[...], preferred_element_type=jnp.float32, precision=lax.Precision.HIGHEST)
        + jnp.dot(g3_ref[...] * inv, m03_ref[...], preferred_element_type=jnp.float32, precision=lax.Precision.HIGHEST)
        + jnp.dot(gf_ref[...], m0g_ref[...], preferred_element_type=jnp.float32, precision=lax.Precision.HIGHEST)
        + mb0_ref[...]
    )
    y = jnp.maximum(y * s * bn0g_ref[...] + bn0b_ref[...], 0.0)
    y = jnp.dot(y, mw1_ref[...], preferred_element_type=jnp.float32, precision=lax.Precision.HIGHEST) + mb1_ref[...]
    y = jnp.maximum(y * s * bn1g_ref[...] + bn1b_ref[...], 0.0)
    y = jnp.maximum(
        jnp.dot(y, mw2_ref[...], preferred_element_type=jnp.float32, precision=lax.Precision.HIGHEST) + mb2_ref[...],
        0.0,
    )
    y_ref[...] = (
        jnp.dot(y, mw3_ref[...], preferred_element_type=jnp.float32, precision=lax.Precision.HIGHEST) + mb3_ref[...]
    )


def _head(g0, g1, g2, g3, inv, gf, m00, m01, m02, m03, m0g, mb0, bn0g, bn0b,
          mw1, mb1, bn1g, bn1b, mw2, mb2, mw3, mb3):
    args = (g0, g1, g2, g3, inv, gf, m00, m01, m02, m03, m0g, mb0, bn0g, bn0b,
            mw1, mb1, bn1g, bn1b, mw2, mb2, mw3, mb3)
    return pl.pallas_call(
        _head_body,
        in_specs=[pl.BlockSpec(a.shape, lambda: (0,) * a.ndim) for a in args],
        out_specs=pl.BlockSpec((G, 1), lambda: (0, 0)),
        out_shape=jax.ShapeDtypeStruct((G, 1), jnp.float32),
    )(*args)


def kernel(x, edge_index, edge_attr, batch, graph_feat, enc_node_w, enc_node_b,
           enc_edge_w, enc_edge_b, conv0_w1, conv0_b1, conv0_w2, conv0_b2,
           gn0_w, gn0_b, gn0_a, conv1_w1, conv1_b1, conv1_w2, conv1_b2, gn1_w,
           gn1_b, gn1_a, conv2_w1, conv2_b1, conv2_w2, conv2_b2, gn2_w, gn2_b,
           gn2_a, mlp_w0, mlp_b0, bn0_g, bn0_b, mlp_w1, mlp_b1, bn1_g, bn1_b,
           mlp_w2, mlp_b2, mlp_w3, mlp_b3):
    r1 = lambda v: v.reshape(1, -1)
    h0 = _linear(x, enc_node_w, r1(enc_node_b), 1000)
    perm = [0] * H
    for q in range(4):
        for i in range(16):
            perm[32 * q + 2 * i] = 32 * q + i
            perm[32 * q + 2 * i + 1] = 32 * q + 16 + i
    perm = jnp.asarray(perm, jnp.int32)
    ea_bf = _linear(edge_attr, enc_edge_w[:, perm], r1(enc_edge_b[perm]),
                    2000, out_dtype=jnp.bfloat16)
    ea = lax.bitcast_convert_type(
        ea_bf.reshape(E, H // 2, 2), jnp.int32
    )
    src2 = edge_index[0].reshape(NW, NG, GRP, C)
    dst2 = edge_index[1].reshape(NW, NG, GRP, C)
    onehot_t, inv = _build_onehot(batch)

    layers = (
        (conv0_w1, conv0_b1, conv0_w2, conv0_b2, gn0_w, gn0_b, gn0_a),
        (conv1_w1, conv1_b1, conv1_w2, conv1_b2, gn1_w, gn1_b, gn1_a),
        (conv2_w1, conv2_b1, conv2_w2, conv2_b2, gn2_w, gn2_b, gn2_a),
    )
    h = h0
    hs = [h0]
    for (w1, b1, w2, b2, gw, gb, ga) in layers:
        a0, a1 = _get_mp_kernel()(h, ea, src2, dst2)
        t, p, q = _layer_passA(h, a0, a1, w1, r1(b1), w2, r1(b2), onehot_t)
        h = _layer_passB(t, h, onehot_t, p, q, inv, r1(gw), r1(gb), r1(ga))
        hs.append(h)

    g0, g1, g2, g3 = _pool(hs[0], hs[1], hs[2], hs[3], onehot_t)
    y = _head(g0, g1, g2, g3, inv, graph_feat,
              mlp_w0[0:H], mlp_w0[H:2 * H], mlp_w0[2 * H:3 * H],
              mlp_w0[3 * H:4 * H], mlp_w0[4 * H:],
              r1(mlp_b0), r1(bn0_g), r1(bn0_b), mlp_w1, r1(mlp_b1), r1(bn1_g),
              r1(bn1_b), mlp_w2, r1(mlp_b2), mlp_w3, r1(mlp_b3))
    return y.reshape(-1)


# speedup vs baseline: 1.6489x; 1.6489x over previous
import functools

import jax
import jax.numpy as jnp
from jax import lax
from jax.experimental import pallas as pl
from jax.experimental.pallas import tpu as pltpu
from jax.experimental.pallas import tpu_sc as plsc

N = 10000
E = 320000
H = 128
G = 64


def _lin_body(x_ref, w_ref, b_ref, o_ref):
    o_ref[...] = (
        jnp.dot(x_ref[...], w_ref[...], preferred_element_type=jnp.float32, precision=lax.Precision.HIGHEST)
        + b_ref[...]
    ).astype(o_ref.dtype)


def _linear(x, w, b2d, block_rows, out_dtype=jnp.float32):
    n, k = x.shape
    m = w.shape[1]
    return pl.pallas_call(
        _lin_body,
        grid=(n // block_rows,),
        in_specs=[
            pl.BlockSpec((block_rows, k), lambda i: (i, 0)),
            pl.BlockSpec((k, m), lambda i: (0, 0)),
            pl.BlockSpec((1, m), lambda i: (0, 0)),
        ],
        out_specs=pl.BlockSpec((block_rows, m), lambda i: (i, 0)),
        out_shape=jax.ShapeDtypeStruct((n, m), out_dtype),
    )(x, w, b2d)


def _ea_body(x_ref, w_ref, b_ref, o_ref):
    t = (
        jnp.dot(x_ref[...], w_ref[...], preferred_element_type=jnp.float32,
                precision=lax.Precision.HIGHEST)
        + b_ref[...]
    ).astype(jnp.bfloat16)
    half = t.shape[0] // 2
    lo = lax.bitcast_convert_type(t[:half], jnp.uint16).astype(jnp.int32)
    hi = lax.bitcast_convert_type(t[half:], jnp.uint16).astype(jnp.int32)
    o_ref[...] = lo | (hi << 16)


def _edge_encoder(x, w, b2d):
    br = 2000
    return pl.pallas_call(
        _ea_body,
        grid=(E // br,),
        in_specs=[
            pl.BlockSpec((br, 16), lambda i: (i, 0)),
            pl.BlockSpec((16, H), lambda i: (0, 0)),
            pl.BlockSpec((1, H), lambda i: (0, 0)),
        ],
        out_specs=pl.BlockSpec((br // 2, H), lambda i: (i, 0)),
        out_shape=jax.ShapeDtypeStruct((E // 2, H), jnp.int32),
    )(x, w, b2d)


_DN0 = (((0,), (0,)), ((), ()))


def _segsum(ot_blk, v):
    return lax.dot_general(ot_blk, v, _DN0, preferred_element_type=jnp.float32, precision=lax.Precision.HIGHEST)


def _onehot_body(bc_ref, ot_ref, inv_ref):
    bc = bc_ref[...]
    ot = (
        jnp.broadcast_to(bc, (N, G)) == lax.broadcasted_iota(jnp.int32, (N, G), 1)
    ).astype(jnp.float32)
    ot_ref[...] = ot
    cnt = _segsum(ot, jnp.ones((N, H), jnp.float32))
    inv_ref[...] = 1.0 / jnp.maximum(cnt, 1.0)


def _build_onehot(batch):
    return pl.pallas_call(
        _onehot_body,
        in_specs=[pl.BlockSpec((N, 1), lambda: (0, 0))],
        out_specs=[
            pl.BlockSpec((N, G), lambda: (0, 0)),
            pl.BlockSpec((G, H), lambda: (0, 0)),
        ],
        out_shape=[
            jax.ShapeDtypeStruct((N, G), jnp.float32),
            jax.ShapeDtypeStruct((G, H), jnp.float32),
        ],
    )(batch.reshape(N, 1))


C = 80
NW = 32
EPW = E // NW
RPT = EPW // C
NG = 25
GRP = RPT // NG
RZ = 16
NZCH = N // RZ

@functools.lru_cache(maxsize=None)
def _get_mp_kernel():
    mesh = plsc.VectorSubcoreMesh(core_axis_name="c", subcore_axis_name="s")
    return functools.partial(
        pl.kernel,
        out_type=(
            jax.ShapeDtypeStruct((N, H), jnp.float32),
            jax.ShapeDtypeStruct((N, H), jnp.float32),
        ),
        mesh=mesh,
        scratch_types=[
            pltpu.VMEM((GRP, C), jnp.int32),
            pltpu.VMEM((GRP, C), jnp.int32),
            pltpu.VMEM((C, H), jnp.float32),
            pltpu.VMEM((C, H), jnp.float32),
            pltpu.VMEM((C // 2, H), jnp.int32),
            pltpu.VMEM((C // 2, H), jnp.int32),
            pltpu.VMEM((RZ, H), jnp.float32),
            pltpu.VMEM_SHARED((N, H), jnp.float32),
            pltpu.SemaphoreType.DMA,
            pltpu.SemaphoreType.DMA,
            pltpu.SemaphoreType.DMA,
            pltpu.SemaphoreType.DMA,
            pltpu.SemaphoreType.DMA,
            pltpu.SemaphoreType.DMA,
        ],
    )(_mp_body)


def _mp_body(h_hbm, ea_hbm, src_hbm, dst_hbm, out0, out1,
             srcv, dstv, hb0, hb1, eb0, eb1, zbuf, acc,
             gs0, gs1, es0, es1, ss0, ss1):
    c = lax.axis_index("c")
    s = lax.axis_index("s")
    wid = c * 16 + s

    def zrow(r, carry):
        for k in range(H // 16):
            zbuf[r, pl.ds(k * 16, 16)] = jnp.zeros((16,), jnp.float32)
        return carry

    lax.fori_loop(0, RZ, zrow, 0)
    for k in range((NZCH + 15) // 16):
        idx = k * 16 + s

        @pl.when(idx < NZCH)
        def _():
            pltpu.sync_copy(zbuf, acc.at[pl.ds(idx * RZ, RZ)])

    plsc.subcore_barrier()

    ebase = wid * EPW

    HC = C // 2

    def group(g, carry):
        pltpu.sync_copy(src_hbm.at[wid, g], srcv)
        pltpu.sync_copy(dst_hbm.at[wid, g], dstv)
        eb_g = wid * (EPW // 2) + g * GRP * HC
        pltpu.async_copy(h_hbm.at[srcv.at[0]], hb0, gs0)
        pltpu.async_copy(ea_hbm.at[pl.ds(eb_g, HC)], eb0, es0)

        def step(t, cy):
            even = lax.rem(t, 2) == 0

            def stage(hbp, ebp, gsp, esp, ssp, hbn, ebn, gsn, esn, ssn):
                @pl.when(t + 1 < GRP)
                def _():
                    @pl.when(t >= 1)
                    def _():
                        pltpu.make_async_copy(
                            hbn, acc.at[dstv.at[t - 1]], ssn
                        ).wait()

                    pltpu.async_copy(h_hbm.at[srcv.at[t + 1]], hbn, gsn)
                    pltpu.async_copy(
                        ea_hbm.at[pl.ds(eb_g + (t + 1) * HC, HC)], ebn, esn
                    )

                pltpu.make_async_copy(h_hbm.at[srcv.at[t]], hbp, gsp).wait()
                pltpu.make_async_copy(
                    ea_hbm.at[pl.ds(eb_g + t * HC, HC)], ebp, esp
                ).wait()

                sixteen = jnp.full((16,), 16, jnp.int32)
                himask = jnp.full((16,), -65536, jnp.int32)

                def rowfn(pr, cz):
                    for q in range(H // 16):
                        w = ebp[pr, pl.ds(q * 16, 16)]
                        a = lax.bitcast_convert_type(
                            lax.shift_left(w, sixteen), jnp.float32)
                        b = lax.bitcast_convert_type(
                            lax.bitwise_and(w, himask), jnp.float32)
                        sl = pl.ds(q * 16, 16)
                        hbp[pr, sl] = jnp.maximum(hbp[pr, sl] + a, 0.0)
                        hbp[HC + pr, sl] = jnp.maximum(hbp[HC + pr, sl] + b, 0.0)
                    return cz

                lax.fori_loop(0, HC, rowfn, 0)
                pltpu.async_copy(hbp, acc.at[dstv.at[t]], ssp, add=True)

            @pl.when(even)
            def _():
                stage(hb0, eb0, gs0, es0, ss0, hb1, eb1, gs1, es1, ss1)

            @pl.when(jnp.logical_not(even))
            def _():
                stage(hb1, eb1, gs1, es1, ss1, hb0, eb0, gs0, es0, ss0)

            return cy

        lax.fori_loop(0, GRP, step, 0)
        pltpu.make_async_copy(hb1, acc.at[dstv.at[GRP - 2]], ss1).wait()
        pltpu.make_async_copy(hb0, acc.at[dstv.at[GRP - 1]], ss0).wait()
        return carry

    lax.fori_loop(0, NG, group, 0)
    plsc.subcore_barrier()

    for k in range((NZCH + 15) // 16):
        idx = k * 16 + s

        @pl.when(idx < NZCH)
        def _():
            pltpu.sync_copy(acc.at[pl.ds(idx * RZ, RZ)], zbuf)

            @pl.when(c == 0)
            def _():
                pltpu.sync_copy(zbuf, out0.at[pl.ds(idx * RZ, RZ)])

            @pl.when(c == 1)
            def _():
                pltpu.sync_copy(zbuf, out1.at[pl.ds(idx * RZ, RZ)])


def _passA_body(h_ref, a0_ref, a1_ref, w1_ref, b1_ref, w2_ref, b2_ref, ot_ref,
                t_ref, p_ref, q_ref):
    i = pl.program_id(0)
    z0 = h_ref[...] + a0_ref[...] + a1_ref[...]
    u = jnp.maximum(
        jnp.dot(z0, w1_ref[...], preferred_element_type=jnp.float32, precision=lax.Precision.HIGHEST) + b1_ref[...],
        0.0,
    )
    t = jnp.dot(u, w2_ref[...], preferred_element_type=jnp.float32, precision=lax.Precision.HIGHEST) + b2_ref[...]
    t_ref[...] = t

    @pl.when(i == 0)
    def _():
        p_ref[...] = jnp.zeros_like(p_ref)
        q_ref[...] = jnp.zeros_like(q_ref)

    ob = ot_ref[...]
    p_ref[...] += _segsum(ob, t)
    q_ref[...] += _segsum(ob, t * t)


def _layer_passA(h, a0, a1, w1, b1, w2, b2, onehot_t):
    br = 1000
    return pl.pallas_call(
        _passA_body,
        grid=(N // br,),
        in_specs=[
            pl.BlockSpec((br, H), lambda i: (i, 0)),
            pl.BlockSpec((br, H), lambda i: (i, 0)),
            pl.BlockSpec((br, H), lambda i: (i, 0)),
            pl.BlockSpec((H, H), lambda i: (0, 0)),
            pl.BlockSpec((1, H), lambda i: (0, 0)),
            pl.BlockSpec((H, H), lambda i: (0, 0)),
            pl.BlockSpec((1, H), lambda i: (0, 0)),
            pl.BlockSpec((br, G), lambda i: (i, 0)),
        ],
        out_specs=[
            pl.BlockSpec((br, H), lambda i: (i, 0)),
            pl.BlockSpec((G, H), lambda i: (0, 0)),
            pl.BlockSpec((G, H), lambda i: (0, 0)),
        ],
        out_shape=[
            jax.ShapeDtypeStruct((N, H), jnp.float32),
            jax.ShapeDtypeStruct((G, H), jnp.float32),
            jax.ShapeDtypeStruct((G, H), jnp.float32),
        ],
    )(h, a0, a1, w1, b1, w2, b2, onehot_t)


def _passB_body(t_ref, hin_ref, ot_ref, p_ref, q_ref, inv_ref, gw_ref, gb_ref,
                ga_ref, out_ref):
    ga = ga_ref[...]
    s1 = p_ref[...] * inv_ref[...]
    s2 = q_ref[...] * inv_ref[...]
    var = s2 - (2.0 * ga - ga * ga) * s1 * s1
    otb = ot_ref[...]
    mean_b = jnp.dot(otb, s1, preferred_element_type=jnp.float32, precision=lax.Precision.HIGHEST)
    var_b = jnp.dot(otb, var, preferred_element_type=jnp.float32, precision=lax.Precision.HIGHEST)
    o = t_ref[...] - mean_b * ga
    z = gw_ref[...] * o * lax.rsqrt(var_b + 1e-5) + gb_ref[...]
    out_ref[...] = jnp.maximum(z, 0.0) + hin_ref[...]


def _layer_passB(t, h_in, ot, p, q, inv, gw, gb, ga):
    br = 1000
    return pl.pallas_call(
        _passB_body,
        grid=(N // br,),
        in_specs=[
            pl.BlockSpec((br, H), lambda i: (i, 0)),
            pl.BlockSpec((br, H), lambda i: (i, 0)),
            pl.BlockSpec((br, G), lambda i: (i, 0)),
            pl.BlockSpec((G, H), lambda i: (0, 0)),
            pl.BlockSpec((G, H), lambda i: (0, 0)),
            pl.BlockSpec((G, H), lambda i: (0, 0)),
            pl.BlockSpec((1, H), lambda i: (0, 0)),
            pl.BlockSpec((1, H), lambda i: (0, 0)),
            pl.BlockSpec((1, H), lambda i: (0, 0)),
        ],
        out_specs=pl.BlockSpec((br, H), lambda i: (i, 0)),
        out_shape=jax.ShapeDtypeStruct((N, H), jnp.float32),
    )(t, h_in, ot, p, q, inv, gw, gb, ga)


def _pool_body(h0_ref, h1_ref, h2_ref, h3_ref, ot_ref, g0_ref, g1_ref, g2_ref,
               g3_ref):
    i = pl.program_id(0)

    @pl.when(i == 0)
    def _():
        g0_ref[...] = jnp.zeros_like(g0_ref)
        g1_ref[...] = jnp.zeros_like(g1_ref)
        g2_ref[...] = jnp.zeros_like(g2_ref)
        g3_ref[...] = jnp.zeros_like(g3_ref)

    ob = ot_ref[...]
    g0_ref[...] += _segsum(ob, h0_ref[...])
    g1_ref[...] += _segsum(ob, h1_ref[...])
    g2_ref[...] += _segsum(ob, h2_ref[...])
    g3_ref[...] += _segsum(ob, h3_ref[...])


def _pool(h0, h1, h2, h3, onehot_t):
    br = 1000
    return pl.pallas_call(
        _pool_body,
        grid=(N // br,),
        in_specs=[pl.BlockSpec((br, H), lambda i: (i, 0))] * 4
        + [pl.BlockSpec((br, G), lambda i: (i, 0))],
        out_specs=[pl.BlockSpec((G, H), lambda i: (0, 0))] * 4,
        out_shape=[jax.ShapeDtypeStruct((G, H), jnp.float32)] * 4,
    )(h0, h1, h2, h3, onehot_t)


def _head_body(g0_ref, g1_ref, g2_ref, g3_ref, inv_ref, gf_ref, m00_ref,
               m01_ref, m02_ref, m03_ref, m0g_ref, mb0_ref, bn0g_ref, bn0b_ref,
               mw1_ref, mb1_ref, bn1g_ref, bn1b_ref, mw2_ref, mb2_ref, mw3_ref,
               mb3_ref, y_ref):
    s = (1.0 + 1e-5) ** -0.5
    inv = inv_ref[...]
    y = (
        jnp.dot(g0_ref[...] * inv, m00_ref[...], preferred_element_type=jnp.float32, precision=lax.Precision.HIGHEST)
        + jnp.dot(g1_ref[...] * inv, m01_ref[...], preferred_element_type=jnp.float32, precision=lax.Precision.HIGHEST)
        + jnp.dot(g2_ref[...] * inv, m02_ref[...], preferred_element_type=jnp.float32, precision=lax.Precision.HIGHEST)
        + jnp.dot(g3_ref[...] * inv, m03_ref[...], preferred_element_type=jnp.float32, precision=lax.Precision.HIGHEST)
        + jnp.dot(gf_ref[...], m0g_ref[...], preferred_element_type=jnp.float32, precision=lax.Precision.HIGHEST)
        + mb0_ref[...]
    )
    y = jnp.maximum(y * s * bn0g_ref[...] + bn0b_ref[...], 0.0)
    y = jnp.dot(y, mw1_ref[...], preferred_element_type=jnp.float32, precision=lax.Precision.HIGHEST) + mb1_ref[...]
    y = jnp.maximum(y * s * bn1g_ref[...] + bn1b_ref[...], 0.0)
    y = jnp.maximum(
        jnp.dot(y, mw2_ref[...], preferred_element_type=jnp.float32, precision=lax.Precision.HIGHEST) + mb2_ref[...],
        0.0,
    )
    y_ref[...] = (
        jnp.dot(y, mw3_ref[...], preferred_element_type=jnp.float32, precision=lax.Precision.HIGHEST) + mb3_ref[...]
    )


def _head(g0, g1, g2, g3, inv, gf, m00, m01, m02, m03, m0g, mb0, bn0g, bn0b,
          mw1, mb1, bn1g, bn1b, mw2, mb2, mw3, mb3):
    args = (g0, g1, g2, g3, inv, gf, m00, m01, m02, m03, m0g, mb0, bn0g, bn0b,
            mw1, mb1, bn1g, bn1b, mw2, mb2, mw3, mb3)
    return pl.pallas_call(
        _head_body,
        in_specs=[pl.BlockSpec(a.shape, lambda: (0,) * a.ndim) for a in args],
        out_specs=pl.BlockSpec((G, 1), lambda: (0, 0)),
        out_shape=jax.ShapeDtypeStruct((G, 1), jnp.float32),
    )(*args)


def kernel(x, edge_index, edge_attr, batch, graph_feat, enc_node_w, enc_node_b,
           enc_edge_w, enc_edge_b, conv0_w1, conv0_b1, conv0_w2, conv0_b2,
           gn0_w, gn0_b, gn0_a, conv1_w1, conv1_b1, conv1_w2, conv1_b2, gn1_w,
           gn1_b, gn1_a, conv2_w1, conv2_b1, conv2_w2, conv2_b2, gn2_w, gn2_b,
           gn2_a, mlp_w0, mlp_b0, bn0_g, bn0_b, mlp_w1, mlp_b1, bn1_g, bn1_b,
           mlp_w2, mlp_b2, mlp_w3, mlp_b3):
    r1 = lambda v: v.reshape(1, -1)
    h0 = _linear(x, enc_node_w, r1(enc_node_b), 1000)
    ea = _edge_encoder(edge_attr, enc_edge_w, r1(enc_edge_b))
    order = (
        jnp.arange(E, dtype=jnp.int32)
        .reshape(E // 2000, 2, 25, C // 2)
        .transpose(0, 2, 1, 3)
        .reshape(-1)
    )
    src2 = edge_index[0][order].reshape(NW, NG, GRP, C)
    dst2 = edge_index[1][order].reshape(NW, NG, GRP, C)
    onehot_t, inv = _build_onehot(batch)

    layers = (
        (conv0_w1, conv0_b1, conv0_w2, conv0_b2, gn0_w, gn0_b, gn0_a),
        (conv1_w1, conv1_b1, conv1_w2, conv1_b2, gn1_w, gn1_b, gn1_a),
        (conv2_w1, conv2_b1, conv2_w2, conv2_b2, gn2_w, gn2_b, gn2_a),
    )
    h = h0
    hs = [h0]
    for (w1, b1, w2, b2, gw, gb, ga) in layers:
        a0, a1 = _get_mp_kernel()(h, ea, src2, dst2)
        t, p, q = _layer_passA(h, a0, a1, w1, r1(b1), w2, r1(b2), onehot_t)
        h = _layer_passB(t, h, onehot_t, p, q, inv, r1(gw), r1(gb), r1(ga))
        hs.append(h)

    g0, g1, g2, g3 = _pool(hs[0], hs[1], hs[2], hs[3], onehot_t)
    y = _head(g0, g1, g2, g3, inv, graph_feat,
              mlp_w0[0:H], mlp_w0[H:2 * H], mlp_w0[2 * H:3 * H],
              mlp_w0[3 * H:4 * H], mlp_w0[4 * H:],
              r1(mlp_b0), r1(bn0_g), r1(bn0_b), mlp_w1, r1(mlp_b1), r1(bn1_g),
              r1(bn1_b), mlp_w2, r1(mlp_b2), mlp_w3, r1(mlp_b3))
    return y.reshape(-1)
